# baseline (speedup 1.0000x reference)
import functools

import jax
import jax.numpy as jnp
from jax import lax
from jax.experimental import pallas as pl
from jax.experimental.pallas import tpu as pltpu
from jax.experimental.pallas import tpu_sc as plsc

N = 39040
F = 128
E = 624640
NGRAPH = N // F

NC = 2
NS = 16
L = 16

NR = 4
RSZ = N // NR
PASSES = NR // NC
PAD_ROWS = 2 * L
EPT = E // NS
CH = 9760
NCH = EPT // CH
G = 128
CCH = 80
NCPY = RSZ // CCH
SEL = 16384
RB = SEL // G
BE = 2048


NGMAX = 6 * ((EPT + 6 * G - 1) // (6 * G))
NGPT = NGMAX + BE // G


def _init_acc(h, acc, sem, s, lo):
    for k in range((NCPY + NS - 1) // NS):
        ci = s + k * NS

        @pl.when(ci < NCPY)
        def _(ci=ci):
            rb = pl.multiple_of(ci * CCH, 8)
            pltpu.async_copy(h.at[pl.ds(pl.multiple_of(lo + rb, 8), CCH)],
                             acc.at[pl.ds(rb, CCH)], sem)

    for k in range((NCPY + NS - 1) // NS):
        ci = s + k * NS

        @pl.when(ci < NCPY)
        def _(ci=ci):
            rb = pl.multiple_of(ci * CCH, 8)
            pltpu.make_async_copy(
                h.at[pl.ds(pl.multiple_of(lo + rb, 8), CCH)],
                acc.at[pl.ds(rb, CCH)], sem).wait()

    plsc.subcore_barrier()


def _copy_out(out, acc, sem, s, lo):
    for k in range((NCPY + NS - 1) // NS):
        ci = s + k * NS

        @pl.when(ci < NCPY)
        def _(ci=ci):
            rb = pl.multiple_of(ci * CCH, 8)
            pltpu.async_copy(acc.at[pl.ds(rb, CCH)],
                             out.at[pl.ds(pl.multiple_of(lo + rb, 8), CCH)],
                             sem)

    for k in range((NCPY + NS - 1) // NS):
        ci = s + k * NS

        @pl.when(ci < NCPY)
        def _(ci=ci):
            rb = pl.multiple_of(ci * CCH, 8)
            pltpu.make_async_copy(
                acc.at[pl.ds(rb, CCH)],
                out.at[pl.ds(pl.multiple_of(lo + rb, 8), CCH)], sem).wait()

    plsc.subcore_barrier()


def _scan_body(src, dst, pairs_list, counts, ring,
               src_chunk, dst_chunk, cnt_stage):
    c = lax.axis_index("c")
    s = lax.axis_index("s")
    wid = s * NC + c
    zero_v = jnp.zeros((L,), jnp.int32)
    one_v = jnp.full((L,), 1, jnp.int32)
    last_v = jnp.full((L,), L - 1, jnp.int32)

    for p in range(PASSES):
        r = c * PASSES + p
        lo = r * RSZ
        slotg = (r * NS + s) * NGPT
        ebase = s * EPT

        def spill(hi, sc):
            nb = (hi - sc + BE - 1) // BE

            def bbody(i, sc):
                rrow = (sc // G) & (RB - 1)
                pltpu.sync_copy(
                    ring.at[pl.ds(rrow, BE // G)],
                    pairs_list.at[pl.ds(slotg + sc // G, BE // G)])
                return sc + BE

            return lax.fori_loop(0, nb, bbody, sc)

        def chunk_body(ci, carry):
            offv, sc = carry
            eb = pl.multiple_of(ebase + ci * CH, 8)
            pltpu.sync_copy(src.at[pl.ds(eb, CH)], src_chunk)
            pltpu.sync_copy(dst.at[pl.ds(eb, CH)], dst_chunk)

            def vec_body(j, offv):
                vs = src_chunk[pl.ds(j * L, L)]
                vd = dst_chunk[pl.ds(j * L, L)]
                m = (vd >= lo) & (vd < lo + RSZ)
                mi = m.astype(jnp.int32)
                ecum = offv + plsc.cumsum(mi)
                e = (ecum - 1) & (SEL - 1)
                plsc.store_scatter(ring, [e >> 7, zero_v, e & (G - 1)], vs,
                                   mask=m)
                plsc.store_scatter(ring, [e >> 7, one_v, e & (G - 1)],
                                   vd - lo, mask=m)
                return lax.gather(
                    ecum, last_v[:, None],
                    lax.GatherDimensionNumbers(offset_dims=(),
                                               collapsed_slice_dims=(0,),
                                               start_index_map=(0,)),
                    (1,), mode=lax.GatherScatterMode.PROMISE_IN_BOUNDS)

            offv = lax.fori_loop(0, CH // L, vec_body, offv)
            off = jnp.max(offv)
            return offv, spill(off - (off & (BE - 1)), sc)

        offv, sc = lax.fori_loop(0, NCH, chunk_body,
                                 (jnp.zeros((L,), jnp.int32), jnp.int32(0)))
        off = jnp.max(offv)

        pad_s = wid * L + lax.iota(jnp.int32, L)
        pad_d = RSZ + lax.iota(jnp.int32, L)
        ng = 6 * ((off + 6 * G - 1) // (6 * G))

        @pl.when(off < ng * G)
        def _():
            for k in range(6 * G // L):
                e = (off + k * L + lax.iota(jnp.int32, L)) & (SEL - 1)
                plsc.store_scatter(ring, [e >> 7, zero_v, e & (G - 1)], pad_s)
                plsc.store_scatter(ring, [e >> 7, one_v, e & (G - 1)],
                                   pad_d + (k % 2) * L)

        spill(ng * G, sc)

        cnt_stage[...] = jnp.broadcast_to(ng, (L,))
        pltpu.sync_copy(cnt_stage,
                        counts.at[pl.ds(pl.multiple_of((r * NS + s) * L, 8),
                                        L)])


def _apply_body(h, pairs_list, counts, out, acc,
                p0, p1, p2, p3, rows0, rows1, rows2, cnt_stage,
                sg0, sg1, sg2, ss0, ss1, ss2, sl0, sl1, sl2, sl3):
    c = lax.axis_index("c")
    s = lax.axis_index("s")
    pairs = (p0, p1, p2, p3)
    rows = (rows0, rows1, rows2)
    sem_g = (sg0, sg1, sg2)
    sem_s = (ss0, ss1, ss2)
    sem_l = (sl0, sl1, sl2, sl3)

    for p in range(PASSES):
        r = c * PASSES + p
        lo = r * RSZ
        slotg = (r * NS + s) * NGPT

        _init_acc(h, acc, sg0, s, lo)

        pltpu.sync_copy(
            counts.at[pl.ds(pl.multiple_of((r * NS + s) * L, 8), L)],
            cnt_stage)
        ng = jnp.max(cnt_stage[...])

        @pl.when(ng > 0)
        def _():
            pltpu.sync_copy(pairs_list.at[slotg], pairs[0])
            pltpu.async_copy(h.at[pairs[0].at[0]], rows[0], sem_g[0])

        @pl.when(ng > 1)
        def _():
            pltpu.sync_copy(pairs_list.at[slotg + 1], pairs[1])
            pltpu.async_copy(h.at[pairs[1].at[0]], rows[1], sem_g[1])

        @pl.when(ng > 2)
        def _():
            pltpu.async_copy(pairs_list.at[slotg + 2], pairs[2], sem_l[2])

        def block_body(ib, carry):
            for k in range(12):
                g = ib * 12 + k
                kr, kp = k % 3, k % 4
                km1r, km1p = (k + 2) % 3, (k + 3) % 4
                k2r, k2p = (k + 2) % 3, (k + 2) % 4
                k3p = (k + 3) % 4

                @pl.when(g < ng)
                def _(g=g, kr=kr, kp=kp, km1r=km1r, km1p=km1p, k2r=k2r,
                      k2p=k2p, k3p=k3p):
                    @pl.when(g >= 1)
                    def _():
                        pltpu.make_async_copy(
                            rows[km1r], acc.at[pairs[km1p].at[1]],
                            sem_s[km1r]).wait()

                    @pl.when(g + 2 < ng)
                    def _():
                        pltpu.make_async_copy(
                            pairs_list.at[slotg + g + 2], pairs[k2p],
                            sem_l[k2p]).wait()
                        pltpu.async_copy(h.at[pairs[k2p].at[0]], rows[k2r],
                                         sem_g[k2r])

                    @pl.when(g + 3 < ng)
                    def _():
                        pltpu.async_copy(pairs_list.at[slotg + g + 3],
                                         pairs[k3p], sem_l[k3p])

                    pltpu.make_async_copy(h.at[pairs[kp].at[0]], rows[kr],
                                          sem_g[kr]).wait()
                    pltpu.async_copy(rows[kr], acc.at[pairs[kp].at[1]],
                                     sem_s[kr], add=True)
            return carry

        lax.fori_loop(0, (ng + 11) // 12, block_body, jnp.int32(0))

        @pl.when(ng > 0)
        def _():
            pltpu.make_async_copy(rows[2], acc.at[pairs[1].at[1]],
                                  sem_s[2]).wait()

        plsc.subcore_barrier()

        _copy_out(out, acc, sg0, s, lo)


_SC_MESH = plsc.VectorSubcoreMesh(core_axis_name="c", subcore_axis_name="s",
                                  num_cores=NC, num_subcores=NS)

_scan_edges = functools.partial(
    pl.kernel,
    out_type=(
        jax.ShapeDtypeStruct((NR * NS * NGPT, 2, G), jnp.int32),
        jax.ShapeDtypeStruct((NR * NS * L,), jnp.int32),
    ),
    mesh=_SC_MESH,
    scratch_types=[
        pltpu.VMEM((RB, 2, G), jnp.int32),
        pltpu.VMEM((CH,), jnp.int32),
        pltpu.VMEM((CH,), jnp.int32),
        pltpu.VMEM((L,), jnp.int32),
    ],
    compiler_params=pltpu.CompilerParams(needs_layout_passes=False),
)(_scan_body)

_aggregate_apply = functools.partial(
    pl.kernel,
    out_type=jax.ShapeDtypeStruct((N, F), jnp.float32),
    mesh=_SC_MESH,
    scratch_types=[
        pltpu.VMEM_SHARED((RSZ + PAD_ROWS, F), jnp.float32),
        pltpu.VMEM((2, G), jnp.int32),
        pltpu.VMEM((2, G), jnp.int32),
        pltpu.VMEM((2, G), jnp.int32),
        pltpu.VMEM((2, G), jnp.int32),
        pltpu.VMEM((G, F), jnp.float32),
        pltpu.VMEM((G, F), jnp.float32),
        pltpu.VMEM((G, F), jnp.float32),
        pltpu.VMEM((L,), jnp.int32),
        pltpu.SemaphoreType.DMA,
        pltpu.SemaphoreType.DMA,
        pltpu.SemaphoreType.DMA,
        pltpu.SemaphoreType.DMA,
        pltpu.SemaphoreType.DMA,
        pltpu.SemaphoreType.DMA,
        pltpu.SemaphoreType.DMA,
        pltpu.SemaphoreType.DMA,
        pltpu.SemaphoreType.DMA,
        pltpu.SemaphoreType.DMA,
    ],
    compiler_params=pltpu.CompilerParams(needs_layout_passes=False),
)(_apply_body)


BLK = 2440


def _conv_block(z_ref, wa_ref, ba_ref, wb_ref, bb_ref, o_ref):
    z = z_ref[...]
    t = jnp.maximum(
        jnp.dot(z, wa_ref[...], preferred_element_type=jnp.float32)
        + ba_ref[...], 0.0)
    o_ref[...] = jnp.maximum(
        jnp.dot(t, wb_ref[...], preferred_element_type=jnp.float32)
        + bb_ref[...], 0.0)


def _conv(z, wa, ba, wb, bb):
    return pl.pallas_call(
        _conv_block,
        grid=(N // BLK,),
        in_specs=[
            pl.BlockSpec((BLK, F), lambda i: (i, 0)),
            pl.BlockSpec((F, F), lambda i: (0, 0)),
            pl.BlockSpec((1, F), lambda i: (0, 0)),
            pl.BlockSpec((F, F), lambda i: (0, 0)),
            pl.BlockSpec((1, F), lambda i: (0, 0)),
        ],
        out_specs=pl.BlockSpec((BLK, F), lambda i: (i, 0)),
        out_shape=jax.ShapeDtypeStruct((N, F), jnp.float32),
    )(z, wa, ba.reshape(1, F), wb, bb.reshape(1, F))


KCH = 2048
BN_SCALE = 1.0 / (1.0 + 1e-5) ** 0.5


def _head_block(hf_ref, w1_ref, bf1_ref, gamma_ref, beta_ref, w2_ref, bf2_ref,
                o_ref, acc_ref):
    k = pl.program_id(0)

    @pl.when(k == 0)
    def _():
        acc_ref[...] = jnp.zeros_like(acc_ref)

    acc_ref[...] += jnp.dot(hf_ref[...], w1_ref[...],
                            preferred_element_type=jnp.float32)

    @pl.when(k == pl.num_programs(0) - 1)
    def _():
        o = acc_ref[...] + bf1_ref[...]
        o = o * (BN_SCALE * gamma_ref[...]) + beta_ref[...]
        o = jnp.maximum(o, 0.0)
        o_ref[...] = (jnp.dot(o, w2_ref[...],
                              preferred_element_type=jnp.float32)
                      + bf2_ref[...])


def _head(hf, w1, bf1, gamma, beta, w2, bf2):
    kd = F * F
    return pl.pallas_call(
        _head_block,
        grid=(kd // KCH,),
        in_specs=[
            pl.BlockSpec((NGRAPH, KCH), lambda k: (0, k)),
            pl.BlockSpec((KCH, F), lambda k: (k, 0)),
            pl.BlockSpec((1, F), lambda k: (0, 0)),
            pl.BlockSpec((1, F), lambda k: (0, 0)),
            pl.BlockSpec((1, F), lambda k: (0, 0)),
            pl.BlockSpec((F, 2), lambda k: (0, 0)),
            pl.BlockSpec((1, 2), lambda k: (0, 0)),
        ],
        out_specs=pl.BlockSpec((NGRAPH, 2), lambda k: (0, 0)),
        out_shape=jax.ShapeDtypeStruct((NGRAPH, 2), jnp.float32),
        scratch_shapes=[pltpu.VMEM((NGRAPH, F), jnp.float32)],
    )(hf, w1, bf1.reshape(1, F), gamma.reshape(1, F), beta.reshape(1, F),
      w2, bf2.reshape(1, 2))


def kernel(x, edge_index, W1a, b1a, W1b, b1b, W2a, b2a, W2b, b2b,
           Wf1, bf1, gamma, beta, Wf2, bf2):
    src = edge_index[0]
    dst = edge_index[1]
    pairs_list, counts = _scan_edges(src, dst)
    z1 = _aggregate_apply(x, pairs_list, counts)
    h1 = _conv(z1, W1a, b1a, W1b, b1b)
    z2 = _aggregate_apply(h1, pairs_list, counts)
    h2 = _conv(z2, W2a, b2a, W2b, b2b)
    hf = h2.reshape(NGRAPH, F * F)
    return _head(hf, Wf1, bf1, gamma, beta, Wf2, bf2)


# speedup vs baseline: 1.0128x; 1.0128x over previous
import functools

import jax
import jax.numpy as jnp
from jax import lax
from jax.experimental import pallas as pl
from jax.experimental.pallas import tpu as pltpu
from jax.experimental.pallas import tpu_sc as plsc

N = 39040
F = 128
E = 624640
NGRAPH = N // F

NC = 2
NS = 16
L = 16

NR = 4
RSZ = N // NR
PASSES = NR // NC
PAD_ROWS = 2 * L
EPT = E // NS
CH = 9760
NCH = EPT // CH
G = 128
CCH = 80
NCPY = RSZ // CCH
SEL = 16384
RB = SEL // G
BE = 2048


NGMAX = 6 * ((EPT + 6 * G - 1) // (6 * G))
NGPT = NGMAX + BE // G


def _init_acc(h, acc, sem, s, lo):
    for k in range((NCPY + NS - 1) // NS):
        ci = s + k * NS

        @pl.when(ci < NCPY)
        def _(ci=ci):
            rb = pl.multiple_of(ci * CCH, 8)
            pltpu.async_copy(h.at[pl.ds(pl.multiple_of(lo + rb, 8), CCH)],
                             acc.at[pl.ds(rb, CCH)], sem)

    for k in range((NCPY + NS - 1) // NS):
        ci = s + k * NS

        @pl.when(ci < NCPY)
        def _(ci=ci):
            rb = pl.multiple_of(ci * CCH, 8)
            pltpu.make_async_copy(
                h.at[pl.ds(pl.multiple_of(lo + rb, 8), CCH)],
                acc.at[pl.ds(rb, CCH)], sem).wait()

    plsc.subcore_barrier()


def _copy_out(out, acc, sem, s, lo):
    for k in range((NCPY + NS - 1) // NS):
        ci = s + k * NS

        @pl.when(ci < NCPY)
        def _(ci=ci):
            rb = pl.multiple_of(ci * CCH, 8)
            pltpu.async_copy(acc.at[pl.ds(rb, CCH)],
                             out.at[pl.ds(pl.multiple_of(lo + rb, 8), CCH)],
                             sem)

    for k in range((NCPY + NS - 1) // NS):
        ci = s + k * NS

        @pl.when(ci < NCPY)
        def _(ci=ci):
            rb = pl.multiple_of(ci * CCH, 8)
            pltpu.make_async_copy(
                acc.at[pl.ds(rb, CCH)],
                out.at[pl.ds(pl.multiple_of(lo + rb, 8), CCH)], sem).wait()

    plsc.subcore_barrier()


def _scan_body(src, dst, pairs_list, counts, ring,
               src_chunk, dst_chunk, cnt_stage):
    c = lax.axis_index("c")
    s = lax.axis_index("s")
    wid = s * NC + c
    zero_v = jnp.zeros((L,), jnp.int32)
    one_v = jnp.full((L,), 1, jnp.int32)

    for p in range(PASSES):
        r = c * PASSES + p
        lo = r * RSZ
        slotg = (r * NS + s) * NGPT
        ebase = s * EPT

        def spill(hi, sc):
            nb = (hi - sc + BE - 1) // BE

            def bbody(i, sc):
                rrow = (sc // G) & (RB - 1)
                pltpu.sync_copy(
                    ring.at[pl.ds(rrow, BE // G)],
                    pairs_list.at[pl.ds(slotg + sc // G, BE // G)])
                return sc + BE

            return lax.fori_loop(0, nb, bbody, sc)

        def chunk_body(ci, carry):
            off, sc = carry
            eb = pl.multiple_of(ebase + ci * CH, 8)
            pltpu.sync_copy(src.at[pl.ds(eb, CH)], src_chunk)
            pltpu.sync_copy(dst.at[pl.ds(eb, CH)], dst_chunk)

            def vec_body(j, off):
                vs = src_chunk[pl.ds(j * L, L)]
                vd = dst_chunk[pl.ds(j * L, L)]
                m = (vd >= lo) & (vd < lo + RSZ)
                mi = m.astype(jnp.int32)
                e = (off + plsc.cumsum(mi) - 1) & (SEL - 1)
                grow = e >> 7
                lane = e & (G - 1)
                plsc.store_scatter(ring, [grow, zero_v, lane], vs, mask=m)
                plsc.store_scatter(ring, [grow, one_v, lane], vd - lo, mask=m)
                return off + jnp.sum(mi)

            off = lax.fori_loop(0, CH // L, vec_body, off)
            return off, spill(off - (off & (BE - 1)), sc)

        off, sc = lax.fori_loop(0, NCH, chunk_body,
                                (jnp.int32(0), jnp.int32(0)))

        pad_s = wid * L + lax.iota(jnp.int32, L)
        pad_d = RSZ + lax.iota(jnp.int32, L)
        ng = 6 * ((off + 6 * G - 1) // (6 * G))

        @pl.when(off < ng * G)
        def _():
            for k in range(6 * G // L):
                e = (off + k * L + lax.iota(jnp.int32, L)) & (SEL - 1)
                plsc.store_scatter(ring, [e >> 7, zero_v, e & (G - 1)], pad_s)
                plsc.store_scatter(ring, [e >> 7, one_v, e & (G - 1)],
                                   pad_d + (k % 2) * L)

        spill(ng * G, sc)

        cnt_stage[...] = jnp.broadcast_to(ng, (L,))
        pltpu.sync_copy(cnt_stage,
                        counts.at[pl.ds(pl.multiple_of((r * NS + s) * L, 8),
                                        L)])


def _apply_body(h, pairs_list, counts, out, acc,
                p0, p1, p2, p3, rows0, rows1, rows2, cnt_stage,
                sg0, sg1, sg2, ss0, ss1, ss2, sl0, sl1, sl2, sl3, si):
    c = lax.axis_index("c")
    s = lax.axis_index("s")
    pairs = (p0, p1, p2, p3)
    rows = (rows0, rows1, rows2)
    sem_g = (sg0, sg1, sg2)
    sem_s = (ss0, ss1, ss2)
    sem_l = (sl0, sl1, sl2, sl3)

    for p in range(PASSES):
        r = c * PASSES + p
        lo = r * RSZ
        slotg = (r * NS + s) * NGPT

        pltpu.sync_copy(
            counts.at[pl.ds(pl.multiple_of((r * NS + s) * L, 8), L)],
            cnt_stage)
        ng = jnp.max(cnt_stage[...])

        @pl.when(ng > 0)
        def _():
            pltpu.sync_copy(pairs_list.at[slotg], pairs[0])
            pltpu.async_copy(h.at[pairs[0].at[0]], rows[0], sem_g[0])

        @pl.when(ng > 1)
        def _():
            pltpu.sync_copy(pairs_list.at[slotg + 1], pairs[1])
            pltpu.async_copy(h.at[pairs[1].at[0]], rows[1], sem_g[1])

        @pl.when(ng > 2)
        def _():
            pltpu.async_copy(pairs_list.at[slotg + 2], pairs[2], sem_l[2])

        _init_acc(h, acc, si, s, lo)

        def block_body(ib, carry):
            for k in range(12):
                g = ib * 12 + k
                kr, kp = k % 3, k % 4
                km1r, km1p = (k + 2) % 3, (k + 3) % 4
                k2r, k2p = (k + 2) % 3, (k + 2) % 4
                k3p = (k + 3) % 4

                @pl.when(g < ng)
                def _(g=g, kr=kr, kp=kp, km1r=km1r, km1p=km1p, k2r=k2r,
                      k2p=k2p, k3p=k3p):
                    @pl.when(g >= 1)
                    def _():
                        pltpu.make_async_copy(
                            rows[km1r], acc.at[pairs[km1p].at[1]],
                            sem_s[km1r]).wait()

                    @pl.when(g + 2 < ng)
                    def _():
                        pltpu.make_async_copy(
                            pairs_list.at[slotg + g + 2], pairs[k2p],
                            sem_l[k2p]).wait()
                        pltpu.async_copy(h.at[pairs[k2p].at[0]], rows[k2r],
                                         sem_g[k2r])

                    @pl.when(g + 3 < ng)
                    def _():
                        pltpu.async_copy(pairs_list.at[slotg + g + 3],
                                         pairs[k3p], sem_l[k3p])

                    pltpu.make_async_copy(h.at[pairs[kp].at[0]], rows[kr],
                                          sem_g[kr]).wait()
                    pltpu.async_copy(rows[kr], acc.at[pairs[kp].at[1]],
                                     sem_s[kr], add=True)
            return carry

        lax.fori_loop(0, (ng + 11) // 12, block_body, jnp.int32(0))

        @pl.when(ng > 0)
        def _():
            pltpu.make_async_copy(rows[2], acc.at[pairs[1].at[1]],
                                  sem_s[2]).wait()

        plsc.subcore_barrier()

        _copy_out(out, acc, si, s, lo)


_SC_MESH = plsc.VectorSubcoreMesh(core_axis_name="c", subcore_axis_name="s",
                                  num_cores=NC, num_subcores=NS)

_scan_edges = functools.partial(
    pl.kernel,
    out_type=(
        jax.ShapeDtypeStruct((NR * NS * NGPT, 2, G), jnp.int32),
        jax.ShapeDtypeStruct((NR * NS * L,), jnp.int32),
    ),
    mesh=_SC_MESH,
    scratch_types=[
        pltpu.VMEM((RB, 2, G), jnp.int32),
        pltpu.VMEM((CH,), jnp.int32),
        pltpu.VMEM((CH,), jnp.int32),
        pltpu.VMEM((L,), jnp.int32),
    ],
    compiler_params=pltpu.CompilerParams(needs_layout_passes=False),
)(_scan_body)

_aggregate_apply = functools.partial(
    pl.kernel,
    out_type=jax.ShapeDtypeStruct((N, F), jnp.float32),
    mesh=_SC_MESH,
    scratch_types=[
        pltpu.VMEM_SHARED((RSZ + PAD_ROWS, F), jnp.float32),
        pltpu.VMEM((2, G), jnp.int32),
        pltpu.VMEM((2, G), jnp.int32),
        pltpu.VMEM((2, G), jnp.int32),
        pltpu.VMEM((2, G), jnp.int32),
        pltpu.VMEM((G, F), jnp.float32),
        pltpu.VMEM((G, F), jnp.float32),
        pltpu.VMEM((G, F), jnp.float32),
        pltpu.VMEM((L,), jnp.int32),
        pltpu.SemaphoreType.DMA,
        pltpu.SemaphoreType.DMA,
        pltpu.SemaphoreType.DMA,
        pltpu.SemaphoreType.DMA,
        pltpu.SemaphoreType.DMA,
        pltpu.SemaphoreType.DMA,
        pltpu.SemaphoreType.DMA,
        pltpu.SemaphoreType.DMA,
        pltpu.SemaphoreType.DMA,
        pltpu.SemaphoreType.DMA,
        pltpu.SemaphoreType.DMA,
    ],
    compiler_params=pltpu.CompilerParams(needs_layout_passes=False),
)(_apply_body)


BLK = 2440


def _conv_block(z_ref, wa_ref, ba_ref, wb_ref, bb_ref, o_ref):
    z = z_ref[...]
    t = jnp.maximum(
        jnp.dot(z, wa_ref[...], preferred_element_type=jnp.float32)
        + ba_ref[...], 0.0)
    o_ref[...] = jnp.maximum(
        jnp.dot(t, wb_ref[...], preferred_element_type=jnp.float32)
        + bb_ref[...], 0.0)


def _conv(z, wa, ba, wb, bb):
    return pl.pallas_call(
        _conv_block,
        grid=(N // BLK,),
        in_specs=[
            pl.BlockSpec((BLK, F), lambda i: (i, 0)),
            pl.BlockSpec((F, F), lambda i: (0, 0)),
            pl.BlockSpec((1, F), lambda i: (0, 0)),
            pl.BlockSpec((F, F), lambda i: (0, 0)),
            pl.BlockSpec((1, F), lambda i: (0, 0)),
        ],
        out_specs=pl.BlockSpec((BLK, F), lambda i: (i, 0)),
        out_shape=jax.ShapeDtypeStruct((N, F), jnp.float32),
    )(z, wa, ba.reshape(1, F), wb, bb.reshape(1, F))


KCH = 2048
BN_SCALE = 1.0 / (1.0 + 1e-5) ** 0.5


def _head_block(hf_ref, w1_ref, bf1_ref, gamma_ref, beta_ref, w2_ref, bf2_ref,
                o_ref, acc_ref):
    k = pl.program_id(0)

    @pl.when(k == 0)
    def _():
        acc_ref[...] = jnp.zeros_like(acc_ref)

    acc_ref[...] += jnp.dot(hf_ref[...], w1_ref[...],
                            preferred_element_type=jnp.float32)

    @pl.when(k == pl.num_programs(0) - 1)
    def _():
        o = acc_ref[...] + bf1_ref[...]
        o = o * (BN_SCALE * gamma_ref[...]) + beta_ref[...]
        o = jnp.maximum(o, 0.0)
        o_ref[...] = (jnp.dot(o, w2_ref[...],
                              preferred_element_type=jnp.float32)
                      + bf2_ref[...])


def _head(hf, w1, bf1, gamma, beta, w2, bf2):
    kd = F * F
    return pl.pallas_call(
        _head_block,
        grid=(kd // KCH,),
        in_specs=[
            pl.BlockSpec((NGRAPH, KCH), lambda k: (0, k)),
            pl.BlockSpec((KCH, F), lambda k: (k, 0)),
            pl.BlockSpec((1, F), lambda k: (0, 0)),
            pl.BlockSpec((1, F), lambda k: (0, 0)),
            pl.BlockSpec((1, F), lambda k: (0, 0)),
            pl.BlockSpec((F, 2), lambda k: (0, 0)),
            pl.BlockSpec((1, 2), lambda k: (0, 0)),
        ],
        out_specs=pl.BlockSpec((NGRAPH, 2), lambda k: (0, 0)),
        out_shape=jax.ShapeDtypeStruct((NGRAPH, 2), jnp.float32),
        scratch_shapes=[pltpu.VMEM((NGRAPH, F), jnp.float32)],
    )(hf, w1, bf1.reshape(1, F), gamma.reshape(1, F), beta.reshape(1, F),
      w2, bf2.reshape(1, 2))


def kernel(x, edge_index, W1a, b1a, W1b, b1b, W2a, b2a, W2b, b2b,
           Wf1, bf1, gamma, beta, Wf2, bf2):
    src = edge_index[0]
    dst = edge_index[1]
    pairs_list, counts = _scan_edges(src, dst)
    z1 = _aggregate_apply(x, pairs_list, counts)
    h1 = _conv(z1, W1a, b1a, W1b, b1b)
    z2 = _aggregate_apply(h1, pairs_list, counts)
    h2 = _conv(z2, W2a, b2a, W2b, b2b)
    hf = h2.reshape(NGRAPH, F * F)
    return _head(hf, Wf1, bf1, gamma, beta, Wf2, bf2)
